# Initial kernel scaffold; baseline (speedup 1.0000x reference)
import jax
import jax.numpy as jnp
from jax.experimental import pallas as pl


def kernel(tensor, masking):
    raise NotImplementedError("write your pallas kernel here")


# speedup vs baseline: 3.3508x; 3.3508x over previous
import jax
import jax.numpy as jnp
from jax.experimental import pallas as pl
from jax.experimental.pallas import tpu as pltpu

B = 16
S = 2048
F = 32
H = 64
SE = S - 1
NT = SE - H + 1
NJ = 4
TBLK = NT // NJ
DEC_F = 8


def _fork_body(x_ref, m_ref, enc_ref, dec_ref, his_ref, fut_ref):
    j = pl.program_id(1)
    toff = j * TBLK

    @pl.when(j == 0)
    def _():
        enc_ref[0] = x_ref[0, :SE, :24]
        his_ref[0] = m_ref[0, :SE, :]

    dec_parts = [x_ref[0, pl.ds(toff + 1 + w, TBLK), 24:32] for w in range(H)]
    dec_ref[0] = jnp.concatenate(dec_parts, axis=-1)
    fut_parts = [m_ref[0, pl.ds(toff + 1 + w, TBLK), :] for w in range(H)]
    fut_ref[0] = jnp.concatenate(fut_parts, axis=-1)


def kernel(tensor, masking):
    enc, dec2d, his, fut2d = pl.pallas_call(
        _fork_body,
        grid=(B, NJ),
        in_specs=[
            pl.BlockSpec((1, S, F), lambda b, j: (b, 0, 0)),
            pl.BlockSpec((1, S, 1), lambda b, j: (b, 0, 0)),
        ],
        out_specs=[
            pl.BlockSpec((1, SE, 24), lambda b, j: (b, 0, 0)),
            pl.BlockSpec((1, TBLK, H * DEC_F), lambda b, j: (b, j, 0)),
            pl.BlockSpec((1, SE, 1), lambda b, j: (b, 0, 0)),
            pl.BlockSpec((1, TBLK, H), lambda b, j: (b, j, 0)),
        ],
        out_shape=[
            jax.ShapeDtypeStruct((B, SE, 24), jnp.float32),
            jax.ShapeDtypeStruct((B, NT, H * DEC_F), jnp.float32),
            jax.ShapeDtypeStruct((B, SE, 1), jnp.float32),
            jax.ShapeDtypeStruct((B, NT, H), jnp.float32),
        ],
        compiler_params=pltpu.CompilerParams(
            dimension_semantics=("parallel", "parallel"),
        ),
    )(tensor, masking)
    dec = dec2d.reshape(B, NT, H, DEC_F)
    fut = fut2d.reshape(B, NT, H, 1)
    return (enc, dec, his, fut)
